# baseline (speedup 1.0000x reference)
import jax, jax.numpy as jnp
import numpy as np
from jax.experimental import pallas as pl

PTS_SIZE = 128
NUM_CLASSES = 14
KNN = (16, 48, 48, 24)
DOWNSAMPLE = (2, 2, 2)
HIDDEN = 256
TOPK = 16
EPS = 1e-5


def _bn(x):
    return x / np.sqrt(1.0 + EPS)


def _conv1x1(x, W, b):
    return jnp.einsum('bchw,cd->bdhw', x, W) + b[None, :, None, None]


def _st_group(x, k):
    B, C, T, N = x.shape
    tm1 = jnp.concatenate([x[:, :, :1], x[:, :, :-1]], axis=2)
    tp1 = jnp.concatenate([x[:, :, 1:], x[:, :, -1:]], axis=2)
    cand = jnp.concatenate([tm1, x, tp1], axis=3)
    d = jnp.sum((x[:, :3, :, :, None] - cand[:, :3, :, None, :]) ** 2, axis=1)
    idx = jax.lax.top_k(-d, k)[1]
    nb = jnp.take_along_axis(cand[:, :, :, None, :], idx[:, None, :, :, :], axis=4)
    off = nb[:, :4] - x[:, :4, :, :, None]
    ctr = jnp.broadcast_to(x[:, 4:, :, :, None], (B, C - 4, T, N, k))
    return jnp.concatenate([off, ctr, nb[:, 4:]], axis=1)


def _select_ind(g, x, pts):
    w = jnp.max(jnp.sum(g[:, :3] ** 2, axis=1), axis=-1)
    ind = jax.lax.top_k(w, pts)[1]
    ret = jnp.take_along_axis(g, ind[:, None, :, :, None], axis=3)
    ret = ret.reshape(g.shape[0], g.shape[1], -1, g.shape[-1])
    x2 = jnp.take_along_axis(x, ind[:, None, :, :], axis=3)
    return ret, x2, ind


def _motion_block(g, wp, bp, wf, bf, wo, bo):
    pos, fea = g[:, :4], g[:, 4:]
    wgt = jax.nn.relu(_bn(_conv1x1(pos, wp, bp)))
    f = jax.nn.relu(_bn(_conv1x1(fea, wf, bf)))
    return jax.nn.relu(_bn(_conv1x1(wgt * f, wo, bo)))


def _point_lstm(fea, Wl, bl):
    B, C, T, N = fea.shape
    P = jnp.moveaxis(fea[:, :4], 2, 0)
    X = jnp.moveaxis(fea[:, 4:], 2, 0)
    h0 = jnp.zeros((B, HIDDEN, N), fea.dtype)
    c0 = jnp.zeros((B, HIDDEN, N), fea.dtype)

    def step(carry, t_in):
        h_prev, c_prev, p_prev = carry
        p_t, x_t = t_in
        d = jnp.sum((p_t[:, :3, :, None] - p_prev[:, :3, None, :]) ** 2, axis=1)
        idx = jax.lax.top_k(-d, TOPK)[1]
        h_nb = jnp.take_along_axis(h_prev[:, :, None, :], idx[:, None], axis=3)
        c_nb = jnp.take_along_axis(c_prev[:, :, None, :], idx[:, None], axis=3)
        p_nb = jnp.take_along_axis(p_prev[:, :, None, :], idx[:, None], axis=3)
        off = p_nb - p_t[:, :, :, None]
        x_tile = jnp.broadcast_to(x_t[:, :, :, None], (B, C - 4, N, TOPK))
        g_in = jnp.concatenate([off, x_tile, h_nb], axis=1)
        gates = jnp.einsum('bcnk,cd->bdnk', g_in, Wl) + bl[None, :, None, None]
        i, f, o, g = jnp.split(gates, 4, axis=1)
        c_new = jax.nn.sigmoid(f) * c_nb + jax.nn.sigmoid(i) * jnp.tanh(g)
        h_new = jax.nn.sigmoid(o) * jnp.tanh(c_new)
        h_t = jnp.max(h_new, axis=-1)
        c_t = jnp.max(c_new, axis=-1)
        return (h_t, c_t, p_t), h_t

    (_, _, _), hs = jax.lax.scan(step, (h0, c0, P[0]), (P, X))
    return jnp.transpose(hs, (1, 2, 0, 3))


def _noop_pallas(x):
    def body(x_ref, o_ref):
        o_ref[...] = x_ref[...]
    return pl.pallas_call(body, out_shape=jax.ShapeDtypeStruct(x.shape, x.dtype))(x)


def kernel(inputs, w1a, b1a, w1b, b1b, wp2, bp2, wf2, bf2, wo2, bo2, wl, bl,
           wp4, bp4, wf4, bf4, wo4, bo4, w5, b5, w6, b6):
    x = jnp.transpose(inputs, (0, 3, 1, 2))
    x = x[:, :, :, ::x.shape[3] // PTS_SIZE]
    x = x[:, :4]
    B, _, T, N = x.shape
    d = jnp.sum((x[:, :3, :, :, None] - x[:, :3, :, None, :]) ** 2, axis=1)
    idx = jax.lax.top_k(-d, KNN[0])[1]
    nb = jnp.take_along_axis(x[:, :, :, None, :], idx[:, None, :, :, :], axis=4)
    ret1 = jnp.concatenate([nb[:, :3] - x[:, :3, :, :, None], nb[:, 3:4]], axis=1)
    ret1 = ret1.reshape(B, 4, T * N, KNN[0])
    f1 = jax.nn.relu(_bn(_conv1x1(ret1, w1a, b1a)))
    f1 = jax.nn.relu(_bn(_conv1x1(f1, w1b, b1b)))
    f1 = jnp.max(f1, axis=-1).reshape(B, 64, T, N)
    fea1 = jnp.concatenate([x, f1], axis=1)
    pts = N // DOWNSAMPLE[0]
    g2 = _st_group(fea1, KNN[1])
    ret2, x, _ = _select_ind(g2, x, pts)
    f2 = _motion_block(ret2, wp2, bp2, wf2, bf2, wo2, bo2)
    f2 = jnp.max(f2, axis=-1).reshape(B, 128, T, pts)
    fea2 = jnp.concatenate([x, f2], axis=1)
    fea3 = _point_lstm(fea2, wl, bl)
    pts2 = pts // DOWNSAMPLE[1]
    g3 = _st_group(fea2, KNN[2])
    _, x, ind = _select_ind(g3, x, pts2)
    fea3 = jnp.take_along_axis(fea3, jnp.broadcast_to(ind[:, None], (B, HIDDEN, T, pts2)), axis=3)
    pts3 = pts2 // DOWNSAMPLE[2]
    g4 = _st_group(fea3, KNN[3])
    ret4, x, _ = _select_ind(g4, x, pts3)
    f4 = _motion_block(ret4, wp4, bp4, wf4, bf4, wo4, bo4)
    f4 = jnp.max(f4, axis=-1).reshape(B, 512, T, pts3)
    out = jax.nn.relu(_bn(_conv1x1(f4, w5, b5)))
    out = jnp.max(out, axis=(2, 3), keepdims=True)
    out = _bn(out)
    out = _conv1x1(out, w6, b6)
    out = _noop_pallas(out)
    return out.reshape(B, NUM_CLASSES)


# speedup vs baseline: 2.4584x; 2.4584x over previous
import numpy as np
import jax
import jax.numpy as jnp
from jax.experimental import pallas as pl
from jax.experimental.pallas import tpu as pltpu

KNN = (16, 48, 48, 24)
HID = 256
TOPK = 16
NCLS = 14
BIG = np.float32(1e30)
BNS = np.float32(1.0 / np.sqrt(1.0 + 1e-5))
_I = False


def _dists(pos, cand_t, m):
    d = jnp.zeros((pos.shape[0], m), jnp.float32)
    for c in range(3):
        dc = pos[:, c:c + 1] - cand_t[c:c + 1, :]
        d = d + dc * dc
    return d


def _argmin_step(d):
    mn = jnp.min(d, axis=1, keepdims=True)
    ids = jax.lax.broadcasted_iota(jnp.int32, d.shape, 1)
    sel = jnp.min(jnp.where(d <= mn, ids, d.shape[1]), axis=1, keepdims=True)
    oh = ids == sel
    return oh.astype(jnp.float32), mn, jnp.where(oh, BIG, d)


def _select(w, npts):
    n = w.shape[0]
    rids = jax.lax.broadcasted_iota(jnp.int32, (n, 1), 0)
    cols = jax.lax.broadcasted_iota(jnp.int32, (1, npts), 1)

    def body(j, carry):
        wv, s = carry
        mx = jnp.max(wv, axis=0, keepdims=True)
        sel = jnp.min(jnp.where(wv >= mx, rids, n), axis=0, keepdims=True)
        ohc = (rids == sel).astype(jnp.float32)
        s = s + ohc * (cols == j).astype(jnp.float32)
        return jnp.where(rids == sel, -BIG, wv), s

    _, s = jax.lax.fori_loop(0, npts, body, (w, jnp.zeros((n, npts), jnp.float32)))
    return s


def _dot(a, b):
    return jnp.dot(a, b, precision=jax.lax.Precision.HIGHEST,
                   preferred_element_type=jnp.float32)


def _mm(a, b):
    return jnp.dot(a.astype(jnp.bfloat16), b.astype(jnp.bfloat16),
                   preferred_element_type=jnp.float32)


def _dot_t(a, b):
    return jax.lax.dot_general(a, b, (((0,), (0,)), ((), ())),
                               precision=jax.lax.Precision.HIGHEST,
                               preferred_element_type=jnp.float32)


def _k1_body(x_ref, xt_ref, w1a, b1a, w1b, b1b, o_ref):
    xf = x_ref[0, 0]
    xt = xt_ref[0, 0]
    d0 = _dists(xf, xt[0:3], xf.shape[0])

    def body(j, carry):
        d, acc = carry
        oh, _, d = _argmin_step(d)
        nb = _dot(oh, xf)
        ret = jnp.concatenate([nb[:, 0:3] - xf[:, 0:3], nb[:, 3:4]], axis=1)
        h = jax.nn.relu((_mm(ret, w1a[...]) + b1a[...]) * BNS)
        h = jax.nn.relu((_mm(h, w1b[...]) + b1b[...]) * BNS)
        return d, jnp.maximum(acc, h)

    _, f1 = jax.lax.fori_loop(
        0, KNN[0], body, (d0, jnp.full((xf.shape[0], 64), -BIG, jnp.float32)))
    o_ref[0, 0] = jnp.concatenate([xf, f1], axis=1)


def _k2_body(f_t, f_m1, f_p1, ft_m1, ft_t, ft_p1, wp, bp, wf, bf, wo, bo, o_ref):
    ct = f_t[0, 0]
    cand = jnp.concatenate([f_m1[0, 0], ct, f_p1[0, 0]], axis=0)
    cand_t = jnp.concatenate(
        [ft_m1[0, 0][0:3], ft_t[0, 0][0:3], ft_p1[0, 0][0:3]], axis=1)
    d0 = _dists(ct, cand_t, 384)

    def p1(j, carry):
        d, _ = carry
        _, mn, d = _argmin_step(d)
        return d, mn

    _, w = jax.lax.fori_loop(0, KNN[1], p1,
                             (d0, jnp.zeros((ct.shape[0], 1), jnp.float32)))
    s = _select(w, 64)
    fc = _dot_t(s, ct)
    dc = _dot_t(s, d0)

    def p2(j, carry):
        d, acc = carry
        oh, _, d = _argmin_step(d)
        nb = _dot(oh, cand)
        off = nb[:, 0:4] - fc[:, 0:4]
        wgt = jax.nn.relu((_mm(off, wp[...]) + bp[...]) * BNS)
        fin = jnp.concatenate([fc[:, 4:], nb[:, 4:]], axis=1)
        f = jax.nn.relu((_mm(fin, wf[...]) + bf[...]) * BNS)
        h = jax.nn.relu((_mm(wgt * f, wo[...]) + bo[...]) * BNS)
        return d, jnp.maximum(acc, h)

    _, f2 = jax.lax.fori_loop(0, KNN[1], p2,
                              (dc, jnp.full((64, 128), -BIG, jnp.float32)))
    o_ref[0, 0] = jnp.concatenate([fc[:, 0:4], f2], axis=1)


def _k3_body(f_t, f_m1, ft_m1, wl, bl, o_ref, h_ref, c_ref):
    t = pl.program_id(1)

    @pl.when(t == 0)
    def _():
        h_ref[...] = jnp.zeros_like(h_ref)
        c_ref[...] = jnp.zeros_like(c_ref)

    cf = f_t[0, 0]
    pt = cf[:, 0:4]
    xt = cf[:, 4:]
    pp = f_m1[0, 0][:, 0:4]
    d0 = _dists(cf, ft_m1[0, 0][0:3], pp.shape[0])
    hcp = jnp.concatenate([h_ref[...], c_ref[...], pp], axis=1)

    def body(j, carry):
        d, hm, cm = carry
        oh, _, d = _argmin_step(d)
        nball = _dot(oh, hcp)
        hnb = nball[:, 0:HID]
        cnb = nball[:, HID:2 * HID]
        pnb = nball[:, 2 * HID:2 * HID + 4]
        gin = jnp.concatenate([pnb - pt, xt, hnb], axis=1)
        gates = _mm(gin, wl[...]) + bl[...]
        ig = jax.nn.sigmoid(gates[:, 0:HID])
        fg = jax.nn.sigmoid(gates[:, HID:2 * HID])
        og = jax.nn.sigmoid(gates[:, 2 * HID:3 * HID])
        gg = jnp.tanh(gates[:, 3 * HID:])
        cn = fg * cnb + ig * gg
        hn = og * jnp.tanh(cn)
        return d, jnp.maximum(hm, hn), jnp.maximum(cm, cn)

    init = jnp.full((pp.shape[0], HID), -BIG, jnp.float32)
    _, hmax, cmax = jax.lax.fori_loop(0, TOPK, body, (d0, init, init))
    h_ref[...] = hmax
    c_ref[...] = cmax
    o_ref[0, 0] = hmax


def _k4_body(f_t, ft_m1, ft_t, ft_p1, h_t, o_ref):
    ct = f_t[0, 0]
    cand_t = jnp.concatenate(
        [ft_m1[0, 0][0:3], ft_t[0, 0][0:3], ft_p1[0, 0][0:3]], axis=1)
    d0 = _dists(ct, cand_t, 192)

    def p1(j, carry):
        d, _ = carry
        _, mn, d = _argmin_step(d)
        return d, mn

    _, w = jax.lax.fori_loop(0, KNN[2], p1,
                             (d0, jnp.zeros((ct.shape[0], 1), jnp.float32)))
    s = _select(w, 32)
    o_ref[0, 0] = _dot_t(s, h_t[0, 0])


def _k5_body(f_t, f_m1, f_p1, ft_m1, ft_t, ft_p1,
             wp, bp, wf, bf, wo, bo, w5, b5, o_ref):
    ct = f_t[0, 0]
    cand = jnp.concatenate([f_m1[0, 0], ct, f_p1[0, 0]], axis=0)
    cand_t = jnp.concatenate(
        [ft_m1[0, 0][0:3], ft_t[0, 0][0:3], ft_p1[0, 0][0:3]], axis=1)
    d0 = _dists(ct, cand_t, 96)

    def p1(j, carry):
        d, _ = carry
        _, mn, d = _argmin_step(d)
        return d, mn

    _, w = jax.lax.fori_loop(0, KNN[3], p1,
                             (d0, jnp.zeros((ct.shape[0], 1), jnp.float32)))
    s = _select(w, 16)
    fc = _dot_t(s, ct)
    dc = _dot_t(s, d0)

    def p2(j, carry):
        d, acc = carry
        oh, _, d = _argmin_step(d)
        nb = _dot(oh, cand)
        off = nb[:, 0:4] - fc[:, 0:4]
        wgt = jax.nn.relu((_mm(off, wp[...]) + bp[...]) * BNS)
        fin = jnp.concatenate([fc[:, 4:], nb[:, 4:]], axis=1)
        f = jax.nn.relu((_mm(fin, wf[...]) + bf[...]) * BNS)
        h = jax.nn.relu((_mm(wgt * f, wo[...]) + bo[...]) * BNS)
        return d, jnp.maximum(acc, h)

    _, f4 = jax.lax.fori_loop(0, KNN[3], p2,
                              (dc, jnp.full((16, 512), -BIG, jnp.float32)))
    o_ref[0, 0] = jax.nn.relu((_mm(f4, w5[...]) + b5[...]) * BNS)


def _k6_body(x_ref, w6, b6, o_ref):
    m = jnp.max(x_ref[0], axis=0)
    m = jnp.max(m, axis=0, keepdims=True)
    o_ref[0] = _mm(m * BNS, w6[...]) + b6[...]


def _wspec(shape):
    nd = len(shape)
    return pl.BlockSpec(shape, lambda b, t, _n=nd: (0,) * _n)


def _fspec(c, n, dt=0, tmax=16):
    if dt == 0:
        im = lambda b, t: (b, t, 0, 0)
    elif dt < 0:
        im = lambda b, t: (b, jnp.maximum(t - 1, 0), 0, 0)
    else:
        im = lambda b, t: (b, jnp.minimum(t + 1, tmax - 1), 0, 0)
    return pl.BlockSpec((1, 1, n, c), im)


def kernel(inputs, w1a, b1a, w1b, b1b, wp2, bp2, wf2, bf2, wo2, bo2, wl, bl,
           wp4, bp4, wf4, bf4, wo4, bo4, w5, b5, w6, b6):
    B, T, N, _ = inputs.shape
    f32 = jnp.float32
    r = lambda v: v.reshape(1, -1)
    w1a_, b1a_ = w1a, r(b1a)
    w1b_, b1b_ = w1b, r(b1b)
    wp2_, bp2_ = wp2, r(bp2)
    wf2_, bf2_ = wf2, r(bf2)
    wo2_, bo2_ = wo2, r(bo2)
    wp4_, bp4_ = wp4, r(bp4)
    wf4_, bf4_ = wf4, r(bf4)
    wo4_, bo4_ = wo4, r(bo4)
    w5_, b5_ = w5, r(b5)
    w6_ = w6

    grid = (B, T)
    par = pltpu.CompilerParams(dimension_semantics=("parallel", "parallel"))
    seq = pltpu.CompilerParams(dimension_semantics=("arbitrary", "arbitrary"))

    x = inputs
    xt = jnp.transpose(inputs, (0, 1, 3, 2))
    fea1 = pl.pallas_call(
        _k1_body, grid=grid,
        in_specs=[_fspec(4, N), _fspec(N, 4),
                  _wspec((4, 32)), _wspec((1, 32)),
                  _wspec((32, 64)), _wspec((1, 64))],
        out_specs=_fspec(68, N),
        out_shape=jax.ShapeDtypeStruct((B, T, N, 68), f32),
        compiler_params=par, interpret=_I,
    )(x, xt, w1a_, b1a_, w1b_, b1b_)

    fea1t = jnp.transpose(fea1, (0, 1, 3, 2))
    fea2 = pl.pallas_call(
        _k2_body, grid=grid,
        in_specs=[_fspec(68, N), _fspec(68, N, -1, T), _fspec(68, N, 1, T),
                  _fspec(N, 68, -1, T), _fspec(N, 68), _fspec(N, 68, 1, T),
                  _wspec((4, 128)), _wspec((1, 128)),
                  _wspec((128, 128)), _wspec((1, 128)),
                  _wspec((128, 128)), _wspec((1, 128))],
        out_specs=_fspec(132, 64),
        out_shape=jax.ShapeDtypeStruct((B, T, 64, 132), f32),
        compiler_params=par, interpret=_I,
    )(fea1, fea1, fea1, fea1t, fea1t, fea1t, wp2_, bp2_, wf2_, bf2_, wo2_, bo2_)

    fea2t = jnp.transpose(fea2, (0, 1, 3, 2))
    fea3 = pl.pallas_call(
        _k3_body, grid=grid,
        in_specs=[_fspec(132, 64), _fspec(132, 64, -1, T), _fspec(64, 132, -1, T),
                  _wspec((388, 1024)), _wspec((1, 1024))],
        out_specs=_fspec(HID, 64),
        out_shape=jax.ShapeDtypeStruct((B, T, 64, HID), f32),
        scratch_shapes=[pltpu.VMEM((64, HID), f32), pltpu.VMEM((64, HID), f32)],
        compiler_params=seq, interpret=_I,
    )(fea2, fea2, fea2t, wl, r(bl))

    fea3c = pl.pallas_call(
        _k4_body, grid=grid,
        in_specs=[_fspec(132, 64),
                  _fspec(64, 132, -1, T), _fspec(64, 132), _fspec(64, 132, 1, T),
                  _fspec(HID, 64)],
        out_specs=_fspec(HID, 32),
        out_shape=jax.ShapeDtypeStruct((B, T, 32, HID), f32),
        compiler_params=par, interpret=_I,
    )(fea2, fea2t, fea2t, fea2t, fea3)

    fea3ct = jnp.transpose(fea3c, (0, 1, 3, 2))
    fea5 = pl.pallas_call(
        _k5_body, grid=grid,
        in_specs=[_fspec(HID, 32), _fspec(HID, 32, -1, T), _fspec(HID, 32, 1, T),
                  _fspec(32, HID, -1, T), _fspec(32, HID), _fspec(32, HID, 1, T),
                  _wspec((4, 512)), _wspec((1, 512)),
                  _wspec((504, 512)), _wspec((1, 512)),
                  _wspec((512, 512)), _wspec((1, 512)),
                  _wspec((512, 1024)), _wspec((1, 1024))],
        out_specs=_fspec(1024, 16),
        out_shape=jax.ShapeDtypeStruct((B, T, 16, 1024), f32),
        compiler_params=par, interpret=_I,
    )(fea3c, fea3c, fea3c, fea3ct, fea3ct, fea3ct,
      wp4_, bp4_, wf4_, bf4_, wo4_, bo4_, w5_, b5_)

    out = pl.pallas_call(
        _k6_body, grid=(B,),
        in_specs=[pl.BlockSpec((1, T, 16, 1024), lambda b: (b, 0, 0, 0)),
                  pl.BlockSpec((1024, NCLS), lambda b: (0, 0)),
                  pl.BlockSpec((1, NCLS), lambda b: (0, 0))],
        out_specs=pl.BlockSpec((1, 1, NCLS), lambda b: (b, 0, 0)),
        out_shape=jax.ShapeDtypeStruct((B, 1, NCLS), f32),
        compiler_params=pltpu.CompilerParams(dimension_semantics=("arbitrary",)),
        interpret=_I,
    )(fea5, w6_, r(b6))
    return out.reshape(B, NCLS)


# speedup vs baseline: 2.6146x; 1.0635x over previous
import numpy as np
import jax
import jax.numpy as jnp
from jax.experimental import pallas as pl
from jax.experimental.pallas import tpu as pltpu

KNN = (16, 48, 48, 24)
HID = 256
TOPK = 16
NCLS = 14
BIG = np.float32(1e30)
BNS = np.float32(1.0 / np.sqrt(1.0 + 1e-5))
_I = False


def _dists(pos, cand_t, m):
    d = jnp.zeros((pos.shape[0], m), jnp.float32)
    for c in range(3):
        dc = pos[:, c:c + 1] - cand_t[c:c + 1, :]
        d = d + dc * dc
    return d


def _argmin_step(d):
    mn = jnp.min(d, axis=1, keepdims=True)
    ids = jax.lax.broadcasted_iota(jnp.int32, d.shape, 1)
    sel = jnp.min(jnp.where(d <= mn, ids, d.shape[1]), axis=1, keepdims=True)
    oh = ids == sel
    return oh.astype(jnp.float32), mn, jnp.where(oh, BIG, d)


def _select(w, npts):
    n = w.shape[0]
    rids = jax.lax.broadcasted_iota(jnp.int32, (n, 1), 0)
    cols = jax.lax.broadcasted_iota(jnp.int32, (1, npts), 1)

    def body(j, carry):
        wv, s = carry
        mx = jnp.max(wv, axis=0, keepdims=True)
        sel = jnp.min(jnp.where(wv >= mx, rids, n), axis=0, keepdims=True)
        ohc = (rids == sel).astype(jnp.float32)
        s = s + ohc * (cols == j).astype(jnp.float32)
        return jnp.where(rids == sel, -BIG, wv), s

    _, s = jax.lax.fori_loop(0, npts, body, (w, jnp.zeros((n, npts), jnp.float32)))
    return s


def _dot(a, b):
    return jnp.dot(a, b, precision=jax.lax.Precision.HIGHEST,
                   preferred_element_type=jnp.float32)


def _mm(a, b):
    return jnp.dot(a.astype(jnp.bfloat16), b.astype(jnp.bfloat16),
                   preferred_element_type=jnp.float32)


def _dot_t(a, b):
    return jax.lax.dot_general(a, b, (((0,), (0,)), ((), ())),
                               precision=jax.lax.Precision.HIGHEST,
                               preferred_element_type=jnp.float32)


def _k1_body(x_ref, xt_ref, w1a, b1a, w1b, b1b, o_ref):
    xf = x_ref[0, 0]
    xt = xt_ref[0, 0]
    d0 = _dists(xf, xt[0:3], xf.shape[0])

    def body(j, carry):
        d, acc = carry
        oh, _, d = _argmin_step(d)
        nb = _dot(oh, xf)
        ret = jnp.concatenate([nb[:, 0:3] - xf[:, 0:3], nb[:, 3:4]], axis=1)
        h = jax.nn.relu((_mm(ret, w1a[...]) + b1a[...]) * BNS)
        h = jax.nn.relu((_mm(h, w1b[...]) + b1b[...]) * BNS)
        return d, jnp.maximum(acc, h)

    _, f1 = jax.lax.fori_loop(
        0, KNN[0], body, (d0, jnp.full((xf.shape[0], 64), -BIG, jnp.float32)))
    o_ref[0, 0] = jnp.concatenate([xf, f1], axis=1)


def _k2_body(f_t, f_m1, f_p1, ft_m1, ft_t, ft_p1, wp, bp, wf, bf, wo, bo, o_ref):
    ct = f_t[0, 0]
    cand = jnp.concatenate([f_m1[0, 0], ct, f_p1[0, 0]], axis=0)
    cand_t = jnp.concatenate(
        [ft_m1[0, 0][0:3], ft_t[0, 0][0:3], ft_p1[0, 0][0:3]], axis=1)
    d0 = _dists(ct, cand_t, 384)

    def p1(j, carry):
        d, _ = carry
        _, mn, d = _argmin_step(d)
        return d, mn

    _, w = jax.lax.fori_loop(0, KNN[1], p1,
                             (d0, jnp.zeros((ct.shape[0], 1), jnp.float32)))
    s = _select(w, 64)
    fc = _dot_t(s, ct)
    dc = _dot_t(s, d0)

    def p2(j, carry):
        d, acc = carry
        oh, _, d = _argmin_step(d)
        nb = _dot(oh, cand)
        off = nb[:, 0:4] - fc[:, 0:4]
        wgt = jax.nn.relu((_mm(off, wp[...]) + bp[...]) * BNS)
        fin = jnp.concatenate([fc[:, 4:], nb[:, 4:]], axis=1)
        f = jax.nn.relu((_mm(fin, wf[...]) + bf[...]) * BNS)
        h = jax.nn.relu((_mm(wgt * f, wo[...]) + bo[...]) * BNS)
        return d, jnp.maximum(acc, h)

    _, f2 = jax.lax.fori_loop(0, KNN[1], p2,
                              (dc, jnp.full((64, 128), -BIG, jnp.float32)))
    o_ref[0, 0] = jnp.concatenate([fc[:, 0:4], f2], axis=1)


def _k3_body(f_t, f_m1, ft_m1, wl, bl, o_ref, h_ref, c_ref):
    t = pl.program_id(1)

    @pl.when(t == 0)
    def _():
        h_ref[...] = jnp.zeros_like(h_ref)
        c_ref[...] = jnp.zeros_like(c_ref)

    cf = f_t[0, 0]
    pt = cf[:, 0:4]
    xt = cf[:, 4:]
    pp = f_m1[0, 0][:, 0:4]
    n = pp.shape[0]
    d = _dists(cf, ft_m1[0, 0][0:3], n)
    ohs = []
    for _ in range(TOPK):
        oh, _, d = _argmin_step(d)
        ohs.append(oh)
    ohk = jnp.concatenate(ohs, axis=0)
    cp = jnp.concatenate([c_ref[...], pp], axis=1)
    cpnb = _dot(ohk, cp)
    hnb = _mm(ohk, h_ref[...])
    cnb = cpnb[:, 0:HID]
    pnb = cpnb[:, HID:HID + 4]
    tile = lambda a: jnp.broadcast_to(
        a[None], (TOPK,) + a.shape).reshape(TOPK * n, a.shape[1])
    gin = jnp.concatenate([pnb - tile(pt), tile(xt), hnb], axis=1)
    gates = _mm(gin, wl[...]) + bl[...]
    ig = jax.nn.sigmoid(gates[:, 0:HID])
    fg = jax.nn.sigmoid(gates[:, HID:2 * HID])
    og = jax.nn.sigmoid(gates[:, 2 * HID:3 * HID])
    gg = jnp.tanh(gates[:, 3 * HID:])
    cn = fg * cnb + ig * gg
    hn = og * jnp.tanh(cn)
    hmax = jnp.max(hn.reshape(TOPK, n, HID), axis=0)
    cmax = jnp.max(cn.reshape(TOPK, n, HID), axis=0)
    h_ref[...] = hmax
    c_ref[...] = cmax
    o_ref[0, 0] = hmax


def _k4_body(f_t, ft_m1, ft_t, ft_p1, h_t, o_ref):
    ct = f_t[0, 0]
    cand_t = jnp.concatenate(
        [ft_m1[0, 0][0:3], ft_t[0, 0][0:3], ft_p1[0, 0][0:3]], axis=1)
    d0 = _dists(ct, cand_t, 192)

    def p1(j, carry):
        d, _ = carry
        _, mn, d = _argmin_step(d)
        return d, mn

    _, w = jax.lax.fori_loop(0, KNN[2], p1,
                             (d0, jnp.zeros((ct.shape[0], 1), jnp.float32)))
    s = _select(w, 32)
    o_ref[0, 0] = _dot_t(s, h_t[0, 0])


def _k5_body(f_t, f_m1, f_p1, ft_m1, ft_t, ft_p1,
             wp, bp, wf, bf, wo, bo, w5, b5, o_ref):
    ct = f_t[0, 0]
    cand = jnp.concatenate([f_m1[0, 0], ct, f_p1[0, 0]], axis=0)
    cand_t = jnp.concatenate(
        [ft_m1[0, 0][0:3], ft_t[0, 0][0:3], ft_p1[0, 0][0:3]], axis=1)
    d0 = _dists(ct, cand_t, 96)

    def p1(j, carry):
        d, _ = carry
        _, mn, d = _argmin_step(d)
        return d, mn

    _, w = jax.lax.fori_loop(0, KNN[3], p1,
                             (d0, jnp.zeros((ct.shape[0], 1), jnp.float32)))
    s = _select(w, 16)
    fc = _dot_t(s, ct)
    dc = _dot_t(s, d0)

    def p2(j, carry):
        d, acc = carry
        oh, _, d = _argmin_step(d)
        nb = _dot(oh, cand)
        off = nb[:, 0:4] - fc[:, 0:4]
        wgt = jax.nn.relu((_mm(off, wp[...]) + bp[...]) * BNS)
        fin = jnp.concatenate([fc[:, 4:], nb[:, 4:]], axis=1)
        f = jax.nn.relu((_mm(fin, wf[...]) + bf[...]) * BNS)
        h = jax.nn.relu((_mm(wgt * f, wo[...]) + bo[...]) * BNS)
        return d, jnp.maximum(acc, h)

    _, f4 = jax.lax.fori_loop(0, KNN[3], p2,
                              (dc, jnp.full((16, 512), -BIG, jnp.float32)))
    o_ref[0, 0] = jax.nn.relu((_mm(f4, w5[...]) + b5[...]) * BNS)


def _k6_body(x_ref, w6, b6, o_ref):
    m = jnp.max(x_ref[0], axis=0)
    m = jnp.max(m, axis=0, keepdims=True)
    o_ref[0] = _mm(m * BNS, w6[...]) + b6[...]


def _wspec(shape):
    nd = len(shape)
    return pl.BlockSpec(shape, lambda b, t, _n=nd: (0,) * _n)


def _fspec(c, n, dt=0, tmax=16):
    if dt == 0:
        im = lambda b, t: (b, t, 0, 0)
    elif dt < 0:
        im = lambda b, t: (b, jnp.maximum(t - 1, 0), 0, 0)
    else:
        im = lambda b, t: (b, jnp.minimum(t + 1, tmax - 1), 0, 0)
    return pl.BlockSpec((1, 1, n, c), im)


def kernel(inputs, w1a, b1a, w1b, b1b, wp2, bp2, wf2, bf2, wo2, bo2, wl, bl,
           wp4, bp4, wf4, bf4, wo4, bo4, w5, b5, w6, b6):
    B, T, N, _ = inputs.shape
    f32 = jnp.float32
    r = lambda v: v.reshape(1, -1)
    w1a_, b1a_ = w1a, r(b1a)
    w1b_, b1b_ = w1b, r(b1b)
    wp2_, bp2_ = wp2, r(bp2)
    wf2_, bf2_ = wf2, r(bf2)
    wo2_, bo2_ = wo2, r(bo2)
    wp4_, bp4_ = wp4, r(bp4)
    wf4_, bf4_ = wf4, r(bf4)
    wo4_, bo4_ = wo4, r(bo4)
    w5_, b5_ = w5, r(b5)
    w6_ = w6

    grid = (B, T)
    par = pltpu.CompilerParams(dimension_semantics=("parallel", "parallel"))
    seq = pltpu.CompilerParams(dimension_semantics=("arbitrary", "arbitrary"))

    x = inputs
    xt = jnp.transpose(inputs, (0, 1, 3, 2))
    fea1 = pl.pallas_call(
        _k1_body, grid=grid,
        in_specs=[_fspec(4, N), _fspec(N, 4),
                  _wspec((4, 32)), _wspec((1, 32)),
                  _wspec((32, 64)), _wspec((1, 64))],
        out_specs=_fspec(68, N),
        out_shape=jax.ShapeDtypeStruct((B, T, N, 68), f32),
        compiler_params=par, interpret=_I,
    )(x, xt, w1a_, b1a_, w1b_, b1b_)

    fea1t = jnp.transpose(fea1, (0, 1, 3, 2))
    fea2 = pl.pallas_call(
        _k2_body, grid=grid,
        in_specs=[_fspec(68, N), _fspec(68, N, -1, T), _fspec(68, N, 1, T),
                  _fspec(N, 68, -1, T), _fspec(N, 68), _fspec(N, 68, 1, T),
                  _wspec((4, 128)), _wspec((1, 128)),
                  _wspec((128, 128)), _wspec((1, 128)),
                  _wspec((128, 128)), _wspec((1, 128))],
        out_specs=_fspec(132, 64),
        out_shape=jax.ShapeDtypeStruct((B, T, 64, 132), f32),
        compiler_params=par, interpret=_I,
    )(fea1, fea1, fea1, fea1t, fea1t, fea1t, wp2_, bp2_, wf2_, bf2_, wo2_, bo2_)

    fea2t = jnp.transpose(fea2, (0, 1, 3, 2))
    fea3 = pl.pallas_call(
        _k3_body, grid=grid,
        in_specs=[_fspec(132, 64), _fspec(132, 64, -1, T), _fspec(64, 132, -1, T),
                  _wspec((388, 1024)), _wspec((1, 1024))],
        out_specs=_fspec(HID, 64),
        out_shape=jax.ShapeDtypeStruct((B, T, 64, HID), f32),
        scratch_shapes=[pltpu.VMEM((64, HID), f32), pltpu.VMEM((64, HID), f32)],
        compiler_params=seq, interpret=_I,
    )(fea2, fea2, fea2t, wl, r(bl))

    fea3c = pl.pallas_call(
        _k4_body, grid=grid,
        in_specs=[_fspec(132, 64),
                  _fspec(64, 132, -1, T), _fspec(64, 132), _fspec(64, 132, 1, T),
                  _fspec(HID, 64)],
        out_specs=_fspec(HID, 32),
        out_shape=jax.ShapeDtypeStruct((B, T, 32, HID), f32),
        compiler_params=par, interpret=_I,
    )(fea2, fea2t, fea2t, fea2t, fea3)

    fea3ct = jnp.transpose(fea3c, (0, 1, 3, 2))
    fea5 = pl.pallas_call(
        _k5_body, grid=grid,
        in_specs=[_fspec(HID, 32), _fspec(HID, 32, -1, T), _fspec(HID, 32, 1, T),
                  _fspec(32, HID, -1, T), _fspec(32, HID), _fspec(32, HID, 1, T),
                  _wspec((4, 512)), _wspec((1, 512)),
                  _wspec((504, 512)), _wspec((1, 512)),
                  _wspec((512, 512)), _wspec((1, 512)),
                  _wspec((512, 1024)), _wspec((1, 1024))],
        out_specs=_fspec(1024, 16),
        out_shape=jax.ShapeDtypeStruct((B, T, 16, 1024), f32),
        compiler_params=par, interpret=_I,
    )(fea3c, fea3c, fea3c, fea3ct, fea3ct, fea3ct,
      wp4_, bp4_, wf4_, bf4_, wo4_, bo4_, w5_, b5_)

    out = pl.pallas_call(
        _k6_body, grid=(B,),
        in_specs=[pl.BlockSpec((1, T, 16, 1024), lambda b: (b, 0, 0, 0)),
                  pl.BlockSpec((1024, NCLS), lambda b: (0, 0)),
                  pl.BlockSpec((1, NCLS), lambda b: (0, 0))],
        out_specs=pl.BlockSpec((1, 1, NCLS), lambda b: (b, 0, 0)),
        out_shape=jax.ShapeDtypeStruct((B, 1, NCLS), f32),
        compiler_params=pltpu.CompilerParams(dimension_semantics=("arbitrary",)),
        interpret=_I,
    )(fea5, w6_, r(b6))
    return out.reshape(B, NCLS)


# speedup vs baseline: 3.7506x; 1.4345x over previous
import numpy as np
import jax
import jax.numpy as jnp
from jax.experimental import pallas as pl
from jax.experimental.pallas import tpu as pltpu

KNN = (16, 48, 48, 24)
HID = 256
TOPK = 16
NCLS = 14
BIG = np.float32(1e30)
BNS = np.float32(1.0 / np.sqrt(1.0 + 1e-5))
_I = False


def _dists(pos, cand_t, m):
    d = jnp.zeros((pos.shape[0], m), jnp.float32)
    for c in range(3):
        dc = pos[:, c:c + 1] - cand_t[c:c + 1, :]
        d = d + dc * dc
    return d


def _argmin_step(d):
    mn = jnp.min(d, axis=1, keepdims=True)
    ids = jax.lax.broadcasted_iota(jnp.int32, d.shape, 1)
    sel = jnp.min(jnp.where(d <= mn, ids, d.shape[1]), axis=1, keepdims=True)
    oh = ids == sel
    return oh.astype(jnp.float32), mn, jnp.where(oh, BIG, d)


def _select(w, npts):
    n = w.shape[0]
    rids = jax.lax.broadcasted_iota(jnp.int32, (n, 1), 0)
    cols = jax.lax.broadcasted_iota(jnp.int32, (1, npts), 1)

    def body(j, carry):
        wv, s = carry
        mx = jnp.max(wv, axis=0, keepdims=True)
        sel = jnp.min(jnp.where(wv >= mx, rids, n), axis=0, keepdims=True)
        ohc = (rids == sel).astype(jnp.float32)
        s = s + ohc * (cols == j).astype(jnp.float32)
        return jnp.where(rids == sel, -BIG, wv), s

    _, s = jax.lax.fori_loop(0, npts, body, (w, jnp.zeros((n, npts), jnp.float32)))
    return s


def _dot(a, b):
    return jnp.dot(a, b, precision=jax.lax.Precision.HIGHEST,
                   preferred_element_type=jnp.float32)


def _mm(a, b):
    return jnp.dot(a.astype(jnp.bfloat16), b.astype(jnp.bfloat16),
                   preferred_element_type=jnp.float32)


def _dot_t(a, b):
    return jax.lax.dot_general(a, b, (((0,), (0,)), ((), ())),
                               precision=jax.lax.Precision.HIGHEST,
                               preferred_element_type=jnp.float32)


def _tile(a, k):
    return jnp.broadcast_to(a[None], (k,) + a.shape).reshape(k * a.shape[0],
                                                             a.shape[1])


def _k1_body(x_ref, xt_ref, w1a, b1a, w1b, b1b, o_ref):
    xf = x_ref[0, 0]
    xt = xt_ref[0, 0]
    d0 = _dists(xf, xt[0:3], xf.shape[0])

    def body(j, carry):
        d, acc = carry
        oh, _, d = _argmin_step(d)
        nb = _dot(oh, xf)
        ret = jnp.concatenate([nb[:, 0:3] - xf[:, 0:3], nb[:, 3:4]], axis=1)
        h = jax.nn.relu((_mm(ret, w1a[...]) + b1a[...]) * BNS)
        h = jax.nn.relu((_mm(h, w1b[...]) + b1b[...]) * BNS)
        return d, jnp.maximum(acc, h)

    _, f1 = jax.lax.fori_loop(
        0, KNN[0], body, (d0, jnp.full((xf.shape[0], 64), -BIG, jnp.float32)))
    o_ref[0, 0] = jnp.concatenate([xf, f1], axis=1)


def _k2_body(f_t, f_m1, f_p1, ft_m1, ft_t, ft_p1, wp, bp, wf, bf, wo, bo, o_ref):
    ct = f_t[0, 0]
    cand = jnp.concatenate([f_m1[0, 0], ct, f_p1[0, 0]], axis=0)
    cand_t = jnp.concatenate(
        [ft_m1[0, 0][0:3], ft_t[0, 0][0:3], ft_p1[0, 0][0:3]], axis=1)
    d0 = _dists(ct, cand_t, 384)

    def p1(j, carry):
        d, _ = carry
        _, mn, d = _argmin_step(d)
        return d, mn

    _, w = jax.lax.fori_loop(0, KNN[1], p1,
                             (d0, jnp.zeros((ct.shape[0], 1), jnp.float32)))
    s = _select(w, 64)
    fc = _dot_t(s, ct)
    d = _dot_t(s, d0)
    ohs = []
    for _ in range(KNN[1]):
        oh, _, d = _argmin_step(d)
        ohs.append(oh)
    ohk = jnp.concatenate(ohs, axis=0)
    nb = _dot(ohk, cand)
    off = nb[:, 0:4] - _tile(fc[:, 0:4], KNN[1])
    wgt = jax.nn.relu((_mm(off, wp[...]) + bp[...]) * BNS)
    fin = jnp.concatenate([_tile(fc[:, 4:], KNN[1]), nb[:, 4:]], axis=1)
    f = jax.nn.relu((_mm(fin, wf[...]) + bf[...]) * BNS)
    h = jax.nn.relu((_mm(wgt * f, wo[...]) + bo[...]) * BNS)
    f2 = jnp.max(h.reshape(KNN[1], 64, 128), axis=0)
    o_ref[0, 0] = jnp.concatenate([fc[:, 0:4], f2], axis=1)


def _k3_body(f_t, f_m1, ft_m1, wl, bl, o_ref, h_ref, c_ref):
    t = pl.program_id(1)

    @pl.when(t == 0)
    def _():
        h_ref[...] = jnp.zeros_like(h_ref)
        c_ref[...] = jnp.zeros_like(c_ref)

    cf = f_t[0, 0]
    pt = cf[:, 0:4]
    xt = cf[:, 4:]
    pp = f_m1[0, 0][:, 0:4]
    n = pp.shape[0]
    d = _dists(cf, ft_m1[0, 0][0:3], n)
    ohs = []
    for _ in range(TOPK):
        oh, _, d = _argmin_step(d)
        ohs.append(oh)
    ohk = jnp.concatenate(ohs, axis=0)
    cp = jnp.concatenate([c_ref[...], pp], axis=1)
    cpnb = _dot(ohk, cp)
    hnb = _mm(ohk, h_ref[...])
    cnb = cpnb[:, 0:HID]
    pnb = cpnb[:, HID:HID + 4]
    tile = lambda a: jnp.broadcast_to(
        a[None], (TOPK,) + a.shape).reshape(TOPK * n, a.shape[1])
    gin = jnp.concatenate([pnb - tile(pt), tile(xt), hnb], axis=1)
    gates = _mm(gin, wl[...]) + bl[...]
    ig = jax.nn.sigmoid(gates[:, 0:HID])
    fg = jax.nn.sigmoid(gates[:, HID:2 * HID])
    og = jax.nn.sigmoid(gates[:, 2 * HID:3 * HID])
    gg = jnp.tanh(gates[:, 3 * HID:])
    cn = fg * cnb + ig * gg
    hn = og * jnp.tanh(cn)
    hmax = jnp.max(hn.reshape(TOPK, n, HID), axis=0)
    cmax = jnp.max(cn.reshape(TOPK, n, HID), axis=0)
    h_ref[...] = hmax
    c_ref[...] = cmax
    o_ref[0, 0] = hmax


def _k4_body(f_t, ft_m1, ft_t, ft_p1, h_t, o_ref):
    ct = f_t[0, 0]
    cand_t = jnp.concatenate(
        [ft_m1[0, 0][0:3], ft_t[0, 0][0:3], ft_p1[0, 0][0:3]], axis=1)
    d0 = _dists(ct, cand_t, 192)

    def p1(j, carry):
        d, _ = carry
        _, mn, d = _argmin_step(d)
        return d, mn

    _, w = jax.lax.fori_loop(0, KNN[2], p1,
                             (d0, jnp.zeros((ct.shape[0], 1), jnp.float32)))
    s = _select(w, 32)
    o_ref[0, 0] = _dot_t(s, h_t[0, 0])


def _k5_body(f_t, f_m1, f_p1, ft_m1, ft_t, ft_p1,
             wp, bp, wf, bf, wo, bo, w5, b5, o_ref):
    ct = f_t[0, 0]
    cand = jnp.concatenate([f_m1[0, 0], ct, f_p1[0, 0]], axis=0)
    cand_t = jnp.concatenate(
        [ft_m1[0, 0][0:3], ft_t[0, 0][0:3], ft_p1[0, 0][0:3]], axis=1)
    d0 = _dists(ct, cand_t, 96)

    def p1(j, carry):
        d, _ = carry
        _, mn, d = _argmin_step(d)
        return d, mn

    _, w = jax.lax.fori_loop(0, KNN[3], p1,
                             (d0, jnp.zeros((ct.shape[0], 1), jnp.float32)))
    s = _select(w, 16)
    fc = _dot_t(s, ct)
    d = _dot_t(s, d0)
    ohs = []
    for _ in range(KNN[3]):
        oh, _, d = _argmin_step(d)
        ohs.append(oh)
    ohk = jnp.concatenate(ohs, axis=0)
    nb = _dot(ohk, cand)
    off = nb[:, 0:4] - _tile(fc[:, 0:4], KNN[3])
    wgt = jax.nn.relu((_mm(off, wp[...]) + bp[...]) * BNS)
    fin = jnp.concatenate([_tile(fc[:, 4:], KNN[3]), nb[:, 4:]], axis=1)
    f = jax.nn.relu((_mm(fin, wf[...]) + bf[...]) * BNS)
    h = jax.nn.relu((_mm(wgt * f, wo[...]) + bo[...]) * BNS)
    f4 = jnp.max(h.reshape(KNN[3], 16, 512), axis=0)
    o_ref[0, 0] = jax.nn.relu((_mm(f4, w5[...]) + b5[...]) * BNS)


def _k6_body(x_ref, w6, b6, o_ref):
    m = jnp.max(x_ref[0], axis=0)
    m = jnp.max(m, axis=0, keepdims=True)
    o_ref[0] = _mm(m * BNS, w6[...]) + b6[...]


def _wspec(shape):
    nd = len(shape)
    return pl.BlockSpec(shape, lambda b, t, _n=nd: (0,) * _n)


def _fspec(c, n, dt=0, tmax=16):
    if dt == 0:
        im = lambda b, t: (b, t, 0, 0)
    elif dt < 0:
        im = lambda b, t: (b, jnp.maximum(t - 1, 0), 0, 0)
    else:
        im = lambda b, t: (b, jnp.minimum(t + 1, tmax - 1), 0, 0)
    return pl.BlockSpec((1, 1, n, c), im)


def kernel(inputs, w1a, b1a, w1b, b1b, wp2, bp2, wf2, bf2, wo2, bo2, wl, bl,
           wp4, bp4, wf4, bf4, wo4, bo4, w5, b5, w6, b6):
    B, T, N, _ = inputs.shape
    f32 = jnp.float32
    r = lambda v: v.reshape(1, -1)
    w1a_, b1a_ = w1a, r(b1a)
    w1b_, b1b_ = w1b, r(b1b)
    wp2_, bp2_ = wp2, r(bp2)
    wf2_, bf2_ = wf2, r(bf2)
    wo2_, bo2_ = wo2, r(bo2)
    wp4_, bp4_ = wp4, r(bp4)
    wf4_, bf4_ = wf4, r(bf4)
    wo4_, bo4_ = wo4, r(bo4)
    w5_, b5_ = w5, r(b5)
    w6_ = w6

    grid = (B, T)
    par = pltpu.CompilerParams(dimension_semantics=("parallel", "parallel"))
    seq = pltpu.CompilerParams(dimension_semantics=("arbitrary", "arbitrary"))

    x = inputs
    xt = jnp.transpose(inputs, (0, 1, 3, 2))
    fea1 = pl.pallas_call(
        _k1_body, grid=grid,
        in_specs=[_fspec(4, N), _fspec(N, 4),
                  _wspec((4, 32)), _wspec((1, 32)),
                  _wspec((32, 64)), _wspec((1, 64))],
        out_specs=_fspec(68, N),
        out_shape=jax.ShapeDtypeStruct((B, T, N, 68), f32),
        compiler_params=par, interpret=_I,
    )(x, xt, w1a_, b1a_, w1b_, b1b_)

    fea1t = jnp.transpose(fea1, (0, 1, 3, 2))
    fea2 = pl.pallas_call(
        _k2_body, grid=grid,
        in_specs=[_fspec(68, N), _fspec(68, N, -1, T), _fspec(68, N, 1, T),
                  _fspec(N, 68, -1, T), _fspec(N, 68), _fspec(N, 68, 1, T),
                  _wspec((4, 128)), _wspec((1, 128)),
                  _wspec((128, 128)), _wspec((1, 128)),
                  _wspec((128, 128)), _wspec((1, 128))],
        out_specs=_fspec(132, 64),
        out_shape=jax.ShapeDtypeStruct((B, T, 64, 132), f32),
        compiler_params=par, interpret=_I,
    )(fea1, fea1, fea1, fea1t, fea1t, fea1t, wp2_, bp2_, wf2_, bf2_, wo2_, bo2_)

    fea2t = jnp.transpose(fea2, (0, 1, 3, 2))
    fea3 = pl.pallas_call(
        _k3_body, grid=grid,
        in_specs=[_fspec(132, 64), _fspec(132, 64, -1, T), _fspec(64, 132, -1, T),
                  _wspec((388, 1024)), _wspec((1, 1024))],
        out_specs=_fspec(HID, 64),
        out_shape=jax.ShapeDtypeStruct((B, T, 64, HID), f32),
        scratch_shapes=[pltpu.VMEM((64, HID), f32), pltpu.VMEM((64, HID), f32)],
        compiler_params=seq, interpret=_I,
    )(fea2, fea2, fea2t, wl, r(bl))

    fea3c = pl.pallas_call(
        _k4_body, grid=grid,
        in_specs=[_fspec(132, 64),
                  _fspec(64, 132, -1, T), _fspec(64, 132), _fspec(64, 132, 1, T),
                  _fspec(HID, 64)],
        out_specs=_fspec(HID, 32),
        out_shape=jax.ShapeDtypeStruct((B, T, 32, HID), f32),
        compiler_params=par, interpret=_I,
    )(fea2, fea2t, fea2t, fea2t, fea3)

    fea3ct = jnp.transpose(fea3c, (0, 1, 3, 2))
    fea5 = pl.pallas_call(
        _k5_body, grid=grid,
        in_specs=[_fspec(HID, 32), _fspec(HID, 32, -1, T), _fspec(HID, 32, 1, T),
                  _fspec(32, HID, -1, T), _fspec(32, HID), _fspec(32, HID, 1, T),
                  _wspec((4, 512)), _wspec((1, 512)),
                  _wspec((504, 512)), _wspec((1, 512)),
                  _wspec((512, 512)), _wspec((1, 512)),
                  _wspec((512, 1024)), _wspec((1, 1024))],
        out_specs=_fspec(1024, 16),
        out_shape=jax.ShapeDtypeStruct((B, T, 16, 1024), f32),
        compiler_params=par, interpret=_I,
    )(fea3c, fea3c, fea3c, fea3ct, fea3ct, fea3ct,
      wp4_, bp4_, wf4_, bf4_, wo4_, bo4_, w5_, b5_)

    out = pl.pallas_call(
        _k6_body, grid=(B,),
        in_specs=[pl.BlockSpec((1, T, 16, 1024), lambda b: (b, 0, 0, 0)),
                  pl.BlockSpec((1024, NCLS), lambda b: (0, 0)),
                  pl.BlockSpec((1, NCLS), lambda b: (0, 0))],
        out_specs=pl.BlockSpec((1, 1, NCLS), lambda b: (b, 0, 0)),
        out_shape=jax.ShapeDtypeStruct((B, 1, NCLS), f32),
        compiler_params=pltpu.CompilerParams(dimension_semantics=("arbitrary",)),
        interpret=_I,
    )(fea5, w6_, r(b6))
    return out.reshape(B, NCLS)


# speedup vs baseline: 3.7518x; 1.0003x over previous
import numpy as np
import jax
import jax.numpy as jnp
from jax.experimental import pallas as pl
from jax.experimental.pallas import tpu as pltpu

KNN = (16, 48, 48, 24)
HID = 256
TOPK = 16
NCLS = 14
BIG = np.float32(1e30)
BNS = np.float32(1.0 / np.sqrt(1.0 + 1e-5))


def _dists(pos, cand_t, m):
    d = jnp.zeros((pos.shape[0], m), jnp.float32)
    for c in range(3):
        dc = pos[:, c:c + 1] - cand_t[c:c + 1, :]
        d = d + dc * dc
    return d


def _argmin_step(d):
    mn = jnp.min(d, axis=1, keepdims=True)
    ids = jax.lax.broadcasted_iota(jnp.int32, d.shape, 1)
    sel = jnp.min(jnp.where(d <= mn, ids, d.shape[1]), axis=1, keepdims=True)
    oh = ids == sel
    return oh.astype(jnp.float32), mn, jnp.where(oh, BIG, d)


def _select(w, npts):
    n = w.shape[0]
    rids = jax.lax.broadcasted_iota(jnp.int32, (n, 1), 0)
    cols = jax.lax.broadcasted_iota(jnp.int32, (1, npts), 1)

    def body(j, carry):
        wv, s = carry
        mx = jnp.max(wv, axis=0, keepdims=True)
        sel = jnp.min(jnp.where(wv >= mx, rids, n), axis=0, keepdims=True)
        ohc = (rids == sel).astype(jnp.float32)
        s = s + ohc * (cols == j).astype(jnp.float32)
        return jnp.where(rids == sel, -BIG, wv), s

    _, s = jax.lax.fori_loop(0, npts, body, (w, jnp.zeros((n, npts), jnp.float32)))
    return s


def _dot(a, b):
    return jnp.dot(a, b, precision=jax.lax.Precision.HIGHEST,
                   preferred_element_type=jnp.float32)


def _mm(a, b):
    return jnp.dot(a.astype(jnp.bfloat16), b.astype(jnp.bfloat16),
                   preferred_element_type=jnp.float32)


def _dot_t(a, b):
    return jax.lax.dot_general(a, b, (((0,), (0,)), ((), ())),
                               precision=jax.lax.Precision.HIGHEST,
                               preferred_element_type=jnp.float32)


def _tile(a, k):
    return jnp.broadcast_to(a[None], (k,) + a.shape).reshape(k * a.shape[0],
                                                             a.shape[1])


def _k1_body(x_ref, xt_ref, w1a, b1a, w1b, b1b, o_ref):
    xf = x_ref[0, 0]
    xt = xt_ref[0, 0]
    d0 = _dists(xf, xt[0:3], xf.shape[0])

    def body(j, carry):
        d, acc = carry
        oh, _, d = _argmin_step(d)
        nb = _dot(oh, xf)
        ret = jnp.concatenate([nb[:, 0:3] - xf[:, 0:3], nb[:, 3:4]], axis=1)
        h = jax.nn.relu((_mm(ret, w1a[...]) + b1a[...]) * BNS)
        h = jax.nn.relu((_mm(h, w1b[...]) + b1b[...]) * BNS)
        return d, jnp.maximum(acc, h)

    _, f1 = jax.lax.fori_loop(
        0, KNN[0], body, (d0, jnp.full((xf.shape[0], 64), -BIG, jnp.float32)))
    o_ref[0, 0] = jnp.concatenate([xf, f1], axis=1)


def _k2_body(f_t, f_m1, f_p1, ft_m1, ft_t, ft_p1, wp, bp, wf, bf, wo, bo, o_ref):
    ct = f_t[0, 0]
    cand = jnp.concatenate([f_m1[0, 0], ct, f_p1[0, 0]], axis=0)
    cand_t = jnp.concatenate(
        [ft_m1[0, 0][0:3], ft_t[0, 0][0:3], ft_p1[0, 0][0:3]], axis=1)
    d0 = _dists(ct, cand_t, 384)

    def p1(j, carry):
        d, _ = carry
        _, mn, d = _argmin_step(d)
        return d, mn

    _, w = jax.lax.fori_loop(0, KNN[1], p1,
                             (d0, jnp.zeros((ct.shape[0], 1), jnp.float32)))
    s = _select(w, 64)
    fc = _dot_t(s, ct)
    d = _dot_t(s, d0)
    ohs = []
    for _ in range(KNN[1]):
        oh, _, d = _argmin_step(d)
        ohs.append(oh)
    ohk = jnp.concatenate(ohs, axis=0)
    nb = _dot(ohk, cand)
    off = nb[:, 0:4] - _tile(fc[:, 0:4], KNN[1])
    wgt = jax.nn.relu((_mm(off, wp[...]) + bp[...]) * BNS)
    fin = jnp.concatenate([_tile(fc[:, 4:], KNN[1]), nb[:, 4:]], axis=1)
    f = jax.nn.relu((_mm(fin, wf[...]) + bf[...]) * BNS)
    h = jax.nn.relu((_mm(wgt * f, wo[...]) + bo[...]) * BNS)
    f2 = jnp.max(h.reshape(KNN[1], 64, 128), axis=0)
    o_ref[0, 0] = jnp.concatenate([fc[:, 0:4], f2], axis=1)


def _k3_body(f_t, f_m1, ft_m1, wl, bl, o_ref, h_ref, c_ref):
    t = pl.program_id(1)

    @pl.when(t == 0)
    def _():
        h_ref[...] = jnp.zeros_like(h_ref)
        c_ref[...] = jnp.zeros_like(c_ref)

    cf = f_t[0, 0]
    pt = cf[:, 0:4]
    xt = cf[:, 4:]
    pp = f_m1[0, 0][:, 0:4]
    n = pp.shape[0]
    d = _dists(cf, ft_m1[0, 0][0:3], n)
    ohs = []
    for _ in range(TOPK):
        oh, _, d = _argmin_step(d)
        ohs.append(oh)
    ohk = jnp.concatenate(ohs, axis=0)
    cp = jnp.concatenate([c_ref[...], pp], axis=1)
    cpnb = _dot(ohk, cp)
    hnb = _mm(ohk, h_ref[...])
    cnb = cpnb[:, 0:HID]
    pnb = cpnb[:, HID:HID + 4]
    tile = lambda a: jnp.broadcast_to(
        a[None], (TOPK,) + a.shape).reshape(TOPK * n, a.shape[1])
    gin = jnp.concatenate([pnb - tile(pt), tile(xt), hnb], axis=1)
    gates = _mm(gin, wl[...]) + bl[...]
    ig = jax.nn.sigmoid(gates[:, 0:HID])
    fg = jax.nn.sigmoid(gates[:, HID:2 * HID])
    og = jax.nn.sigmoid(gates[:, 2 * HID:3 * HID])
    gg = jnp.tanh(gates[:, 3 * HID:])
    cn = fg * cnb + ig * gg
    hn = og * jnp.tanh(cn)
    hmax = jnp.max(hn.reshape(TOPK, n, HID), axis=0)
    cmax = jnp.max(cn.reshape(TOPK, n, HID), axis=0)
    h_ref[...] = hmax
    c_ref[...] = cmax
    o_ref[0, 0] = hmax


def _k4_body(f_t, ft_m1, ft_t, ft_p1, h_t, o_ref):
    ct = f_t[0, 0]
    cand_t = jnp.concatenate(
        [ft_m1[0, 0][0:3], ft_t[0, 0][0:3], ft_p1[0, 0][0:3]], axis=1)
    d0 = _dists(ct, cand_t, 192)

    def p1(j, carry):
        d, _ = carry
        _, mn, d = _argmin_step(d)
        return d, mn

    _, w = jax.lax.fori_loop(0, KNN[2], p1,
                             (d0, jnp.zeros((ct.shape[0], 1), jnp.float32)))
    s = _select(w, 32)
    o_ref[0, 0] = _dot_t(s, h_t[0, 0])


def _k5_body(f_t, f_m1, f_p1, ft_m1, ft_t, ft_p1,
             wp, bp, wf, bf, wo, bo, w5, b5, o_ref):
    ct = f_t[0, 0]
    cand = jnp.concatenate([f_m1[0, 0], ct, f_p1[0, 0]], axis=0)
    cand_t = jnp.concatenate(
        [ft_m1[0, 0][0:3], ft_t[0, 0][0:3], ft_p1[0, 0][0:3]], axis=1)
    d0 = _dists(ct, cand_t, 96)

    def p1(j, carry):
        d, _ = carry
        _, mn, d = _argmin_step(d)
        return d, mn

    _, w = jax.lax.fori_loop(0, KNN[3], p1,
                             (d0, jnp.zeros((ct.shape[0], 1), jnp.float32)))
    s = _select(w, 16)
    fc = _dot_t(s, ct)
    d = _dot_t(s, d0)
    ohs = []
    for _ in range(KNN[3]):
        oh, _, d = _argmin_step(d)
        ohs.append(oh)
    ohk = jnp.concatenate(ohs, axis=0)
    nb = _dot(ohk, cand)
    off = nb[:, 0:4] - _tile(fc[:, 0:4], KNN[3])
    wgt = jax.nn.relu((_mm(off, wp[...]) + bp[...]) * BNS)
    fin = jnp.concatenate([_tile(fc[:, 4:], KNN[3]), nb[:, 4:]], axis=1)
    f = jax.nn.relu((_mm(fin, wf[...]) + bf[...]) * BNS)
    h = jax.nn.relu((_mm(wgt * f, wo[...]) + bo[...]) * BNS)
    f4 = jnp.max(h.reshape(KNN[3], 16, 512), axis=0)
    o_ref[0, 0] = jax.nn.relu((_mm(f4, w5[...]) + b5[...]) * BNS)


def _k6_body(x_ref, w6, b6, o_ref):
    m = jnp.max(x_ref[0], axis=0)
    m = jnp.max(m, axis=0, keepdims=True)
    o_ref[0] = _mm(m * BNS, w6[...]) + b6[...]


def _wspec(shape):
    nd = len(shape)
    return pl.BlockSpec(shape, lambda b, t, _n=nd: (0,) * _n)


def _fspec(c, n, dt=0, tmax=16):
    if dt == 0:
        im = lambda b, t: (b, t, 0, 0)
    elif dt < 0:
        im = lambda b, t: (b, jnp.maximum(t - 1, 0), 0, 0)
    else:
        im = lambda b, t: (b, jnp.minimum(t + 1, tmax - 1), 0, 0)
    return pl.BlockSpec((1, 1, n, c), im)


def kernel(inputs, w1a, b1a, w1b, b1b, wp2, bp2, wf2, bf2, wo2, bo2, wl, bl,
           wp4, bp4, wf4, bf4, wo4, bo4, w5, b5, w6, b6):
    B, T, N, _ = inputs.shape
    f32 = jnp.float32
    r = lambda v: v.reshape(1, -1)
    w1a_, b1a_ = w1a, r(b1a)
    w1b_, b1b_ = w1b, r(b1b)
    wp2_, bp2_ = wp2, r(bp2)
    wf2_, bf2_ = wf2, r(bf2)
    wo2_, bo2_ = wo2, r(bo2)
    wp4_, bp4_ = wp4, r(bp4)
    wf4_, bf4_ = wf4, r(bf4)
    wo4_, bo4_ = wo4, r(bo4)
    w5_, b5_ = w5, r(b5)
    w6_ = w6

    grid = (B, T)
    par = pltpu.CompilerParams(dimension_semantics=("parallel", "parallel"))
    seq = pltpu.CompilerParams(dimension_semantics=("arbitrary", "arbitrary"))

    x = inputs
    xt = jnp.transpose(inputs, (0, 1, 3, 2))
    fea1 = pl.pallas_call(
        _k1_body, grid=grid,
        in_specs=[_fspec(4, N), _fspec(N, 4),
                  _wspec((4, 32)), _wspec((1, 32)),
                  _wspec((32, 64)), _wspec((1, 64))],
        out_specs=_fspec(68, N),
        out_shape=jax.ShapeDtypeStruct((B, T, N, 68), f32),
        compiler_params=par,
    )(x, xt, w1a_, b1a_, w1b_, b1b_)

    fea1t = jnp.transpose(fea1, (0, 1, 3, 2))
    fea2 = pl.pallas_call(
        _k2_body, grid=grid,
        in_specs=[_fspec(68, N), _fspec(68, N, -1, T), _fspec(68, N, 1, T),
                  _fspec(N, 68, -1, T), _fspec(N, 68), _fspec(N, 68, 1, T),
                  _wspec((4, 128)), _wspec((1, 128)),
                  _wspec((128, 128)), _wspec((1, 128)),
                  _wspec((128, 128)), _wspec((1, 128))],
        out_specs=_fspec(132, 64),
        out_shape=jax.ShapeDtypeStruct((B, T, 64, 132), f32),
        compiler_params=par,
    )(fea1, fea1, fea1, fea1t, fea1t, fea1t, wp2_, bp2_, wf2_, bf2_, wo2_, bo2_)

    fea2t = jnp.transpose(fea2, (0, 1, 3, 2))
    fea3 = pl.pallas_call(
        _k3_body, grid=grid,
        in_specs=[_fspec(132, 64), _fspec(132, 64, -1, T), _fspec(64, 132, -1, T),
                  _wspec((388, 1024)), _wspec((1, 1024))],
        out_specs=_fspec(HID, 64),
        out_shape=jax.ShapeDtypeStruct((B, T, 64, HID), f32),
        scratch_shapes=[pltpu.VMEM((64, HID), f32), pltpu.VMEM((64, HID), f32)],
        compiler_params=seq,
    )(fea2, fea2, fea2t, wl, r(bl))

    fea3c = pl.pallas_call(
        _k4_body, grid=grid,
        in_specs=[_fspec(132, 64),
                  _fspec(64, 132, -1, T), _fspec(64, 132), _fspec(64, 132, 1, T),
                  _fspec(HID, 64)],
        out_specs=_fspec(HID, 32),
        out_shape=jax.ShapeDtypeStruct((B, T, 32, HID), f32),
        compiler_params=par,
    )(fea2, fea2t, fea2t, fea2t, fea3)

    fea3ct = jnp.transpose(fea3c, (0, 1, 3, 2))
    fea5 = pl.pallas_call(
        _k5_body, grid=grid,
        in_specs=[_fspec(HID, 32), _fspec(HID, 32, -1, T), _fspec(HID, 32, 1, T),
                  _fspec(32, HID, -1, T), _fspec(32, HID), _fspec(32, HID, 1, T),
                  _wspec((4, 512)), _wspec((1, 512)),
                  _wspec((504, 512)), _wspec((1, 512)),
                  _wspec((512, 512)), _wspec((1, 512)),
                  _wspec((512, 1024)), _wspec((1, 1024))],
        out_specs=_fspec(1024, 16),
        out_shape=jax.ShapeDtypeStruct((B, T, 16, 1024), f32),
        compiler_params=par,
    )(fea3c, fea3c, fea3c, fea3ct, fea3ct, fea3ct,
      wp4_, bp4_, wf4_, bf4_, wo4_, bo4_, w5_, b5_)

    out = pl.pallas_call(
        _k6_body, grid=(B,),
        in_specs=[pl.BlockSpec((1, T, 16, 1024), lambda b: (b, 0, 0, 0)),
                  pl.BlockSpec((1024, NCLS), lambda b: (0, 0)),
                  pl.BlockSpec((1, NCLS), lambda b: (0, 0))],
        out_specs=pl.BlockSpec((1, 1, NCLS), lambda b: (b, 0, 0)),
        out_shape=jax.ShapeDtypeStruct((B, 1, NCLS), f32),
        compiler_params=pltpu.CompilerParams(dimension_semantics=("arbitrary",)),
    )(fea5, w6_, r(b6))
    return out.reshape(B, NCLS)
